# baseline (speedup 1.0000x reference)
import functools

import jax
import jax.numpy as jnp
from jax import lax
from jax.experimental import pallas as pl
from jax.experimental.pallas import tpu as pltpu
from jax.experimental.pallas import tpu_sc as plsc

N = 1048576
NC, NS, L = 2, 16, 16
NW = NC * NS
E = N // NW
C = 4096
NCHUNK = E // C
CAND = 288
RPN = 256
OB16 = (288 + 16) // 16

_mesh = plsc.VectorSubcoreMesh(
    core_axis_name="c", subcore_axis_name="s", num_cores=NC, num_subcores=NS)


def _softplus(d):
    e = jnp.exp(-jnp.abs(d))
    z = e / (2.0 + e)
    z2 = z * z
    p = 1.0 + z2 * (1.0 / 3.0 + z2 * (1.0 / 5.0 + z2 * (1.0 / 7.0)))
    return jnp.maximum(d, 0.0) + 2.0 * z * p


def _count_ge(buf, nv16, mv):
    def body(t, cnt_v):
        v = buf[pl.ds(t * 16, 16)]
        return cnt_v + jnp.where(v >= mv, 1, 0).astype(jnp.int32)
    cnt_v = lax.fori_loop(0, nv16, body, jnp.zeros((16,), jnp.int32))
    return jnp.sum(cnt_v)


def _bisect(buf, nv16, k):
    def body(_, lohi):
        lo, hi = lohi
        mid = (lo + hi) >> 1
        mv = plsc.bitcast(jnp.full((16,), mid, jnp.int32), jnp.float32)
        c = _count_ge(buf, nv16, mv)
        take = c >= k
        return (jnp.where(take, mid, lo), jnp.where(take, hi, mid))
    lo, hi = lax.fori_loop(0, 31, body, (jnp.int32(0), jnp.int32(0x7F800000)))
    return lo


def _phase1(x0_hbm, x1_hbm, p1_hbm, p2_hbm, rc_hbm, r1_hbm, r2_hbm, gc_hbm,
            vals_out, scal_out,
            bx0a, bx1a, bp1a, bp2a, brca, br1a, br2a, bgca,
            bx0b, bx1b, bp1b, bp2b, brcb, br1b, br2b, bgcb,
            ivbuf, cbuf, obuf, sbuf, semA, semB):
    wid = lax.axis_index("s") * NC + lax.axis_index("c")
    iota = lax.iota(jnp.int32, 16)
    neg1 = jnp.full((16,), -1.0, jnp.float32)

    def fill(t, _):
        cbuf[pl.ds(t * 16, 16)] = neg1
        obuf[pl.ds(jnp.minimum(t, OB16 - 1) * 16, 16)] = neg1
        return 0
    lax.fori_loop(0, (CAND + 16) // 16, fill, 0)

    hbms = (x0_hbm, x1_hbm, p1_hbm, p2_hbm, rc_hbm, r1_hbm, r2_hbm, gc_hbm)
    seta = (bx0a, bx1a, bp1a, bp2a, brca, br1a, br2a, bgca)
    setb = (bx0b, bx1b, bp1b, bp2b, brcb, br1b, br2b, bgcb)

    def fire(j, bufs, sem):
        base = wid * E + j * C
        for h, b in zip(hbms, bufs):
            pltpu.async_copy(h.at[pl.ds(base, C)], b, sem)

    def drain(bufs, sem):
        for h, b in zip(hbms, bufs):
            pltpu.make_async_copy(h.at[pl.ds(0, C)], b, sem).wait()

    def process(bufs, state):
        bx0, bx1, bp1, bp2, brc, br1, br2, bgc = bufs

        def scan(i, kv):
            b1 = i * 16
            g = bgc[pl.ds(b1, 16)]
            rc = brc[pl.ds(b1, 16)]
            inter = (g <= 1) | (rc == 1.0)
            cv = plsc.all_reduce_population_count(inter)
            anyv = cv > 0
            m1 = anyv & (iota == 0)
            s_k = kv[0]
            plsc.store_compressed(ivbuf.at[pl.ds(s_k, 16)], jnp.full((16,), i, jnp.int32), mask=m1)
            return kv + anyv.astype(jnp.int32)
        kv = lax.fori_loop(0, C // 16, scan, jnp.zeros((16,), jnp.int32), unroll=4)
        kk = kv[0]

        def body(t, st):
            off, pos_v, npos_v, nneg_v, regr_v, rcnt_v = st
            vec = ivbuf[pl.ds((t >> 4) * 16, 16)]
            lane = t & 15
            i = jnp.sum(jnp.where(iota == lane, vec, 0))
            b1 = i * 16
            g = bgc[pl.ds(b1, 16)]
            rc = brc[pl.ds(b1, 16)]
            rm = rc == 1.0
            x0 = bx0[pl.ds(b1, 16)]
            x1 = bx1[pl.ds(b1, 16)]
            d = jnp.where(g >= 1, x0 - x1, x1 - x0)
            ce = _softplus(d)
            pos = g == 1
            neg = g == 0
            pos_v = pos_v + jnp.where(pos, ce, 0.0)
            npos_v = npos_v + pos.astype(jnp.float32)
            nneg_v = nneg_v + neg.astype(jnp.float32)
            r1 = br1[pl.ds(b1, 16)]
            r2 = br2[pl.ds(b1, 16)]
            p1 = bp1[pl.ds(b1, 16)]
            p2 = bp2[pl.ds(b1, 16)]
            d1 = jnp.abs(r1 - p1)
            d2 = jnp.abs(r2 - p2)
            l1 = jnp.where(d1 < 1.0 / 9.0, 4.5 * d1 * d1, d1 - 1.0 / 18.0)
            l2 = jnp.where(d2 < 1.0 / 9.0, 4.5 * d2 * d2, d2 - 1.0 / 18.0)
            regr_v = regr_v + jnp.where(rm, l1 + l2, 0.0)
            rcnt_v = rcnt_v + rm.astype(jnp.float32)
            cnt = plsc.all_reduce_population_count(neg)[0]
            plsc.store_compressed(cbuf.at[pl.ds(off, 16)], ce, mask=neg)
            return (off + cnt, pos_v, npos_v, nneg_v, regr_v, rcnt_v)
        return lax.fori_loop(0, kk, body, state)

    def two_chunks(jj, state):
        drain(seta, semA)
        fire(2 * jj + 1, setb, semB)
        state = process(seta, state)
        drain(setb, semB)

        @pl.when(2 * jj + 2 < NCHUNK)
        def _():
            fire(2 * jj + 2, seta, semA)
        return process(setb, state)

    zf = jnp.zeros((16,), jnp.float32)
    fire(0, seta, semA)
    state = lax.fori_loop(0, NCHUNK // 2, two_chunks,
                          (jnp.int32(0), zf, zf, zf, zf, zf))

    c_w, pos_v, npos_v, nneg_v, regr_v, rcnt_v = state
    cbuf[pl.ds(c_w, 16)] = neg1
    nv16 = (c_w + 15) >> 4

    @pl.when(c_w <= RPN)
    def _copy_small():
        def cps(t, _):
            obuf[pl.ds(t * 16, 16)] = cbuf[pl.ds(t * 16, 16)]
            return 0
        lax.fori_loop(0, CAND // 16, cps, 0)

    @pl.when(c_w > RPN)
    def _local_topk():
        tbits = _bisect(cbuf, nv16, RPN)
        tv = plsc.bitcast(jnp.full((16,), tbits, jnp.int32), jnp.float32)

        def compact(t, woff):
            v = cbuf[pl.ds(t * 16, 16)]
            m = v > tv
            cnt = plsc.all_reduce_population_count(m)[0]
            plsc.store_compressed(obuf.at[pl.ds(woff, 16)], v, mask=m)
            return woff + cnt
        g = lax.fori_loop(0, nv16, compact, jnp.int32(0))
        rem = RPN - g

        def app(t, _):
            mask = (iota + t * 16) < rem
            start = jnp.minimum(g + t * 16, jnp.int32(CAND))
            plsc.store_compressed(obuf.at[pl.ds(start, 16)], tv, mask=mask)
            return 0
        lax.fori_loop(0, RPN // 16, app, 0)

    ps = jnp.sum(pos_v)
    nps = jnp.sum(npos_v)
    nns = jnp.sum(nneg_v)
    rs = jnp.sum(regr_v)
    rc_s = jnp.sum(rcnt_v)
    svec = (jnp.where(iota == 0, ps, 0.0) + jnp.where(iota == 1, nps, 0.0)
            + jnp.where(iota == 2, nns, 0.0) + jnp.where(iota == 3, rs, 0.0)
            + jnp.where(iota == 4, rc_s, 0.0))
    sbuf[...] = svec
    pltpu.sync_copy(sbuf, scal_out.at[pl.ds(wid * 16, 16)])
    pltpu.sync_copy(obuf.at[pl.ds(0, CAND)], vals_out.at[pl.ds(wid * CAND, CAND)])


def _phase2(vals_hbm, scal_hbm, out_hbm, vbuf, svbuf, dense, outb):
    wid = lax.axis_index("s") * NC + lax.axis_index("c")

    @pl.when(wid == 0)
    def _run():
        pltpu.sync_copy(scal_hbm, svbuf)
        pltpu.sync_copy(vals_hbm, vbuf)

        def accum(w, acc):
            return acc + svbuf[pl.ds(w * 16, 16)]
        tot = lax.fori_loop(0, NW, accum, jnp.zeros((16,), jnp.float32))
        pos_sum = tot[0]
        num_pos = tot[1].astype(jnp.int32)
        num_neg = tot[2].astype(jnp.int32)
        regr_sum = tot[3]
        rcnt = tot[4]

        def densify(w, woff):
            c_w = svbuf[pl.ds(w * 16, 16)][2].astype(jnp.int32)
            kept = jnp.minimum(c_w, jnp.int32(RPN))
            nv = (kept + 15) >> 4

            def cp(t, _):
                dense[pl.ds(woff + t * 16, 16)] = vbuf[pl.ds(w * CAND + t * 16, 16)]
                return 0
            lax.fori_loop(0, nv, cp, 0)
            return woff + nv * 16
        m = lax.fori_loop(0, NW, densify, jnp.int32(0))
        m16 = m >> 4

        k = jnp.minimum(num_neg, RPN - num_pos)
        kk = jnp.maximum(k, 1)
        tbits = _bisect(dense, m16, kk)
        tv_v = plsc.bitcast(jnp.full((16,), tbits, jnp.int32), jnp.float32)
        tv = tv_v[0]

        def final(t, cs):
            cv, sv = cs
            v = dense[pl.ds(t * 16, 16)]
            mgt = v > tv_v
            return (cv + jnp.where(mgt, 1.0, 0.0), sv + jnp.where(mgt, v, 0.0))
        cv, sv = lax.fori_loop(0, m16, final,
                               (jnp.zeros((16,), jnp.float32),
                                jnp.zeros((16,), jnp.float32)))
        c_gt = jnp.sum(cv)
        s_gt = jnp.sum(sv)
        neg_sum = s_gt + (kk.astype(jnp.float32) - c_gt) * tv
        neg_sum = jnp.where(k > 0, neg_sum, 0.0)

        cls_loss = (pos_sum + neg_sum) * (1.0 / float(RPN))
        rcnt_v = jnp.full((16,), rcnt, jnp.float32)
        regr_v = jnp.full((16,), regr_sum, jnp.float32) / jnp.maximum(rcnt_v, 1.0)
        loss_v = jnp.full((16,), cls_loss, jnp.float32) + jnp.where(
            rcnt_v > 0.0, regr_v, 0.0)
        outb[...] = loss_v
        pltpu.sync_copy(outb, out_hbm)


_k1 = functools.partial(
    pl.kernel,
    out_type=[jax.ShapeDtypeStruct((NW * CAND,), jnp.float32),
              jax.ShapeDtypeStruct((NW * 16,), jnp.float32)],
    mesh=_mesh,
    compiler_params=pltpu.CompilerParams(needs_layout_passes=False),
    scratch_types=(
        ([pltpu.VMEM((C,), jnp.float32)] * 7
         + [pltpu.VMEM((C,), jnp.int32)]) * 2
        + [pltpu.VMEM((C // 16 + 16,), jnp.int32),
           pltpu.VMEM((E + 32, ), jnp.float32),
           pltpu.VMEM((CAND + 16,), jnp.float32),
           pltpu.VMEM((16,), jnp.float32),
           pltpu.SemaphoreType.DMA,
           pltpu.SemaphoreType.DMA]
    ),
)(_phase1)

_k2 = functools.partial(
    pl.kernel,
    out_type=jax.ShapeDtypeStruct((16,), jnp.float32),
    mesh=_mesh,
    compiler_params=pltpu.CompilerParams(needs_layout_passes=False),
    scratch_types=[
        pltpu.VMEM((NW * CAND,), jnp.float32),
        pltpu.VMEM((NW * 16,), jnp.float32),
        pltpu.VMEM((NW * CAND + 16,), jnp.float32),
        pltpu.VMEM((16,), jnp.float32),
    ],
)(_phase2)


def kernel(pred_cls, pred_regr, gt_cls, gt_regr):
    x0 = pred_cls[0, :, 0]
    x1 = pred_cls[0, :, 1]
    p1 = pred_regr[0, :, 0]
    p2 = pred_regr[0, :, 1]
    rc = gt_regr[0, :, 0]
    r1 = gt_regr[0, :, 1]
    r2 = gt_regr[0, :, 2]
    gc = gt_cls.reshape(-1)
    vals, scal = _k1(x0, x1, p1, p2, rc, r1, r2, gc)
    out = _k2(vals, scal)
    return out[0]


# speedup vs baseline: 1.0033x; 1.0033x over previous
import functools

import jax
import jax.numpy as jnp
from jax import lax
from jax.experimental import pallas as pl
from jax.experimental.pallas import tpu as pltpu
from jax.experimental.pallas import tpu_sc as plsc

N = 1048576
NC, NS, L = 2, 16, 16
NW = NC * NS
E = N // NW
C = 4096
NCHUNK = E // C
CAND = 288
RPN = 256
OB16 = (288 + 16) // 16

_mesh = plsc.VectorSubcoreMesh(
    core_axis_name="c", subcore_axis_name="s", num_cores=NC, num_subcores=NS)


def _softplus(d):
    e = jnp.exp(-jnp.abs(d))
    z = e / (2.0 + e)
    z2 = z * z
    p = 1.0 + z2 * (1.0 / 3.0 + z2 * (1.0 / 5.0 + z2 * (1.0 / 7.0)))
    return jnp.maximum(d, 0.0) + 2.0 * z * p


def _count_ge(buf, nv16, mv):
    def body(t, cnt_v):
        v = buf[pl.ds(t * 16, 16)]
        return cnt_v + jnp.where(v >= mv, 1, 0).astype(jnp.int32)
    cnt_v = lax.fori_loop(0, nv16, body, jnp.zeros((16,), jnp.int32))
    return jnp.sum(cnt_v)


def _bisect(buf, nv16, k):
    def body(_, lohi):
        lo, hi = lohi
        mid = (lo + hi) >> 1
        mv = plsc.bitcast(jnp.full((16,), mid, jnp.int32), jnp.float32)
        c = _count_ge(buf, nv16, mv)
        take = c >= k
        return (jnp.where(take, mid, lo), jnp.where(take, hi, mid))
    lo, hi = lax.fori_loop(0, 31, body, (jnp.int32(0), jnp.int32(0x7F800000)))
    return lo


def _phase1(x0_hbm, x1_hbm, p1_hbm, p2_hbm, rc_hbm, r1_hbm, r2_hbm, gc_hbm,
            vals_out, scal_out,
            bx0a, bx1a, bp1a, bp2a, brca, br1a, br2a, bgca,
            bx0b, bx1b, bp1b, bp2b, brcb, br1b, br2b, bgcb,
            ivbuf, cbuf, obuf, sbuf, semA, semB):
    wid = lax.axis_index("s") * NC + lax.axis_index("c")
    iota = lax.iota(jnp.int32, 16)
    neg1 = jnp.full((16,), -1.0, jnp.float32)

    def fill(t, _):
        cbuf[pl.ds(t * 16, 16)] = neg1
        obuf[pl.ds(jnp.minimum(t, OB16 - 1) * 16, 16)] = neg1
        return 0
    lax.fori_loop(0, (CAND + 16) // 16, fill, 0)

    hbms = (x0_hbm, x1_hbm, p1_hbm, p2_hbm, rc_hbm, r1_hbm, r2_hbm, gc_hbm)
    seta = (bx0a, bx1a, bp1a, bp2a, brca, br1a, br2a, bgca)
    setb = (bx0b, bx1b, bp1b, bp2b, brcb, br1b, br2b, bgcb)

    def fire(j, bufs, sem):
        base = wid * E + j * C
        for h, b in zip(hbms, bufs):
            pltpu.async_copy(h.at[pl.ds(base, C)], b, sem)

    def drain(bufs, sem):
        for h, b in zip(hbms, bufs):
            pltpu.make_async_copy(h.at[pl.ds(0, C)], b, sem).wait()

    def process(bufs, state):
        bx0, bx1, bp1, bp2, brc, br1, br2, bgc = bufs

        def scan(i, kv):
            b1 = i * 16
            g = bgc[pl.ds(b1, 16)]
            rc = brc[pl.ds(b1, 16)]
            inter = (g <= 1) | (rc == 1.0)
            cv = plsc.all_reduce_population_count(inter)
            anyv = cv > 0
            m1 = anyv & (iota == 0)
            s_k = kv[0]
            plsc.store_compressed(ivbuf.at[pl.ds(s_k, 16)], jnp.full((16,), i, jnp.int32), mask=m1)
            return kv + anyv.astype(jnp.int32)
        kv = lax.fori_loop(0, C // 16, scan, jnp.zeros((16,), jnp.int32))
        kk = kv[0]

        def body(t, st):
            off, pos_v, npos_v, nneg_v, regr_v, rcnt_v = st
            vec = ivbuf[pl.ds((t >> 4) * 16, 16)]
            lane = t & 15
            i = jnp.sum(jnp.where(iota == lane, vec, 0))
            b1 = i * 16
            g = bgc[pl.ds(b1, 16)]
            rc = brc[pl.ds(b1, 16)]
            rm = rc == 1.0
            x0 = bx0[pl.ds(b1, 16)]
            x1 = bx1[pl.ds(b1, 16)]
            d = jnp.where(g >= 1, x0 - x1, x1 - x0)
            ce = _softplus(d)
            pos = g == 1
            neg = g == 0
            pos_v = pos_v + jnp.where(pos, ce, 0.0)
            npos_v = npos_v + pos.astype(jnp.float32)
            nneg_v = nneg_v + neg.astype(jnp.float32)
            r1 = br1[pl.ds(b1, 16)]
            r2 = br2[pl.ds(b1, 16)]
            p1 = bp1[pl.ds(b1, 16)]
            p2 = bp2[pl.ds(b1, 16)]
            d1 = jnp.abs(r1 - p1)
            d2 = jnp.abs(r2 - p2)
            l1 = jnp.where(d1 < 1.0 / 9.0, 4.5 * d1 * d1, d1 - 1.0 / 18.0)
            l2 = jnp.where(d2 < 1.0 / 9.0, 4.5 * d2 * d2, d2 - 1.0 / 18.0)
            regr_v = regr_v + jnp.where(rm, l1 + l2, 0.0)
            rcnt_v = rcnt_v + rm.astype(jnp.float32)
            cnt = plsc.all_reduce_population_count(neg)[0]
            plsc.store_compressed(cbuf.at[pl.ds(off, 16)], ce, mask=neg)
            return (off + cnt, pos_v, npos_v, nneg_v, regr_v, rcnt_v)
        return lax.fori_loop(0, kk, body, state)

    def two_chunks(jj, state):
        drain(seta, semA)
        fire(2 * jj + 1, setb, semB)
        state = process(seta, state)
        drain(setb, semB)

        @pl.when(2 * jj + 2 < NCHUNK)
        def _():
            fire(2 * jj + 2, seta, semA)
        return process(setb, state)

    zf = jnp.zeros((16,), jnp.float32)
    fire(0, seta, semA)
    state = lax.fori_loop(0, NCHUNK // 2, two_chunks,
                          (jnp.int32(0), zf, zf, zf, zf, zf))

    c_w, pos_v, npos_v, nneg_v, regr_v, rcnt_v = state
    cbuf[pl.ds(c_w, 16)] = neg1
    nv16 = (c_w + 15) >> 4

    @pl.when(c_w <= RPN)
    def _copy_small():
        def cps(t, _):
            obuf[pl.ds(t * 16, 16)] = cbuf[pl.ds(t * 16, 16)]
            return 0
        lax.fori_loop(0, CAND // 16, cps, 0)

    @pl.when(c_w > RPN)
    def _local_topk():
        tbits = _bisect(cbuf, nv16, RPN)
        tv = plsc.bitcast(jnp.full((16,), tbits, jnp.int32), jnp.float32)

        def compact(t, woff):
            v = cbuf[pl.ds(t * 16, 16)]
            m = v > tv
            cnt = plsc.all_reduce_population_count(m)[0]
            plsc.store_compressed(obuf.at[pl.ds(woff, 16)], v, mask=m)
            return woff + cnt
        g = lax.fori_loop(0, nv16, compact, jnp.int32(0))
        rem = RPN - g

        def app(t, _):
            mask = (iota + t * 16) < rem
            start = jnp.minimum(g + t * 16, jnp.int32(CAND))
            plsc.store_compressed(obuf.at[pl.ds(start, 16)], tv, mask=mask)
            return 0
        lax.fori_loop(0, RPN // 16, app, 0)

    ps = jnp.sum(pos_v)
    nps = jnp.sum(npos_v)
    nns = jnp.sum(nneg_v)
    rs = jnp.sum(regr_v)
    rc_s = jnp.sum(rcnt_v)
    svec = (jnp.where(iota == 0, ps, 0.0) + jnp.where(iota == 1, nps, 0.0)
            + jnp.where(iota == 2, nns, 0.0) + jnp.where(iota == 3, rs, 0.0)
            + jnp.where(iota == 4, rc_s, 0.0))
    sbuf[...] = svec
    pltpu.sync_copy(sbuf, scal_out.at[pl.ds(wid * 16, 16)])
    pltpu.sync_copy(obuf.at[pl.ds(0, CAND)], vals_out.at[pl.ds(wid * CAND, CAND)])


def _phase2(vals_hbm, scal_hbm, out_hbm, vbuf, svbuf, dense, outb):
    wid = lax.axis_index("s") * NC + lax.axis_index("c")

    @pl.when(wid == 0)
    def _run():
        pltpu.sync_copy(scal_hbm, svbuf)
        pltpu.sync_copy(vals_hbm, vbuf)

        def accum(w, acc):
            return acc + svbuf[pl.ds(w * 16, 16)]
        tot = lax.fori_loop(0, NW, accum, jnp.zeros((16,), jnp.float32))
        pos_sum = tot[0]
        num_pos = tot[1].astype(jnp.int32)
        num_neg = tot[2].astype(jnp.int32)
        regr_sum = tot[3]
        rcnt = tot[4]

        def densify(w, woff):
            c_w = svbuf[pl.ds(w * 16, 16)][2].astype(jnp.int32)
            kept = jnp.minimum(c_w, jnp.int32(RPN))
            nv = (kept + 15) >> 4

            def cp(t, _):
                dense[pl.ds(woff + t * 16, 16)] = vbuf[pl.ds(w * CAND + t * 16, 16)]
                return 0
            lax.fori_loop(0, nv, cp, 0)
            return woff + nv * 16
        m = lax.fori_loop(0, NW, densify, jnp.int32(0))
        m16 = m >> 4

        k = jnp.minimum(num_neg, RPN - num_pos)
        kk = jnp.maximum(k, 1)
        tbits = _bisect(dense, m16, kk)
        tv_v = plsc.bitcast(jnp.full((16,), tbits, jnp.int32), jnp.float32)
        tv = tv_v[0]

        def final(t, cs):
            cv, sv = cs
            v = dense[pl.ds(t * 16, 16)]
            mgt = v > tv_v
            return (cv + jnp.where(mgt, 1.0, 0.0), sv + jnp.where(mgt, v, 0.0))
        cv, sv = lax.fori_loop(0, m16, final,
                               (jnp.zeros((16,), jnp.float32),
                                jnp.zeros((16,), jnp.float32)))
        c_gt = jnp.sum(cv)
        s_gt = jnp.sum(sv)
        neg_sum = s_gt + (kk.astype(jnp.float32) - c_gt) * tv
        neg_sum = jnp.where(k > 0, neg_sum, 0.0)

        cls_loss = (pos_sum + neg_sum) * (1.0 / float(RPN))
        rcnt_v = jnp.full((16,), rcnt, jnp.float32)
        regr_v = jnp.full((16,), regr_sum, jnp.float32) / jnp.maximum(rcnt_v, 1.0)
        loss_v = jnp.full((16,), cls_loss, jnp.float32) + jnp.where(
            rcnt_v > 0.0, regr_v, 0.0)
        outb[...] = loss_v
        pltpu.sync_copy(outb, out_hbm)


_k1 = functools.partial(
    pl.kernel,
    out_type=[jax.ShapeDtypeStruct((NW * CAND,), jnp.float32),
              jax.ShapeDtypeStruct((NW * 16,), jnp.float32)],
    mesh=_mesh,
    compiler_params=pltpu.CompilerParams(needs_layout_passes=False),
    scratch_types=(
        ([pltpu.VMEM((C,), jnp.float32)] * 7
         + [pltpu.VMEM((C,), jnp.int32)]) * 2
        + [pltpu.VMEM((C // 16 + 16,), jnp.int32),
           pltpu.VMEM((E + 32, ), jnp.float32),
           pltpu.VMEM((CAND + 16,), jnp.float32),
           pltpu.VMEM((16,), jnp.float32),
           pltpu.SemaphoreType.DMA,
           pltpu.SemaphoreType.DMA]
    ),
)(_phase1)

_k2 = functools.partial(
    pl.kernel,
    out_type=jax.ShapeDtypeStruct((16,), jnp.float32),
    mesh=_mesh,
    compiler_params=pltpu.CompilerParams(needs_layout_passes=False),
    scratch_types=[
        pltpu.VMEM((NW * CAND,), jnp.float32),
        pltpu.VMEM((NW * 16,), jnp.float32),
        pltpu.VMEM((NW * CAND + 16,), jnp.float32),
        pltpu.VMEM((16,), jnp.float32),
    ],
)(_phase2)


def kernel(pred_cls, pred_regr, gt_cls, gt_regr):
    x0 = pred_cls[0, :, 0]
    x1 = pred_cls[0, :, 1]
    p1 = pred_regr[0, :, 0]
    p2 = pred_regr[0, :, 1]
    rc = gt_regr[0, :, 0]
    r1 = gt_regr[0, :, 1]
    r2 = gt_regr[0, :, 2]
    gc = gt_cls.reshape(-1)
    vals, scal = _k1(x0, x1, p1, p2, rc, r1, r2, gc)
    out = _k2(vals, scal)
    return out[0]
